# baseline (speedup 1.0000x reference)
import jax
import jax.numpy as jnp
import numpy as np
from jax.experimental import pallas as pl
from jax.experimental.pallas import tpu as pltpu

B, T, DIN, D1, H, C = 2048, 50, 128, 256, 64, 64
LAM, ETA = 0.95, 0.5
BC = 512


def _z_kernel(bx_ref, w1_ref, b1_ref, w2_ref, b2_ref, z_ref):
    x = bx_ref[0]
    s1 = jnp.maximum(
        jnp.dot(x, w1_ref[...], preferred_element_type=jnp.float32)
        + b1_ref[...], 0.0)
    z = jnp.maximum(
        jnp.dot(s1, w2_ref[...], preferred_element_type=jnp.float32)
        + b2_ref[...], 0.0)
    z_ref[0] = z


def _scan_kernel(invh_ref, pow2_ref, zT_ref, wT_ref, cT_ref, g2_ref,
                 bb2_ref, byT_ref, w3T_ref, b3g_ref, w4T_ref, b4g_ref,
                 loss_ref, acc_ref, hT_s, a_s, hnT_s, ehn_s, r_s):
    t = pl.program_id(0)

    @pl.when(t == 0)
    def _():
        hT_s[...] = jnp.zeros_like(hT_s)

        def zero_body(i, _):
            a_s[i] = jnp.zeros((H, B), jnp.float32)
            return 0
        jax.lax.fori_loop(0, H, zero_body, 0)

    zT = zT_ref[0]
    hT_old = hT_s[...]
    zc = jnp.dot(cT_ref[...], zT, preferred_element_type=jnp.float32)
    hn = jnp.maximum(
        jnp.dot(wT_ref[...], hT_old, preferred_element_type=jnp.float32)
        + zc, 0.0)
    hnT_s[...] = hn
    ehn_s[...] = invh_ref[t] * hn
    r_s[...] = jnp.zeros_like(r_s)

    for kt in range(H // 16):
        o = kt * 16
        et16 = ehn_s[o:o + 16, :]
        hv = hnT_s[...]
        acc = None
        for u in range(16):
            i1 = o + u
            ehr = jnp.broadcast_to(et16[u:u + 1, :], (H, B))
            ao = a_s[i1]
            av = ao + ehr * hv
            a_s[i1] = av
            contrib = ehr * av
            acc = contrib if acc is None else acc + contrib
        r_s[...] += acc

    hn2 = hnT_s[...]
    rT = pow2_ref[t] * r_s[...]
    hs = jnp.dot(wT_ref[...], hn2, preferred_element_type=jnp.float32) \
        + zc + rT
    mu = jnp.mean(hs, axis=1, keepdims=True)
    d = hs - mu
    sig = jnp.sqrt(jnp.mean(d * d, axis=1, keepdims=True))
    hs = jnp.maximum(g2_ref[...] * d / sig + bb2_ref[...], 0.0)
    hT_s[...] = hs

    @pl.when(t == T - 1)
    def _():
        hh = jnp.maximum(
            jnp.dot(w3T_ref[...], hs, preferred_element_type=jnp.float32)
            + b3g_ref[...], 0.0)
        logits = jnp.dot(w4T_ref[...], hh, preferred_element_type=jnp.float32) \
            + b4g_ref[...]
        m = jnp.max(logits, axis=0, keepdims=True)
        sh = logits - m
        lse = jnp.log(jnp.sum(jnp.exp(sh), axis=0, keepdims=True))
        lsm = sh - lse
        byT = byT_ref[...]
        per_sample = jnp.sum(-byT * lsm, axis=0, keepdims=True)
        loss_ref[...] = jnp.mean(per_sample, axis=1, keepdims=True)
        iota = jax.lax.broadcasted_iota(jnp.int32, (C, B), 0)
        pred = jnp.min(jnp.where(logits == m, iota, C), axis=0, keepdims=True)
        mby = jnp.max(byT, axis=0, keepdims=True)
        lab = jnp.min(jnp.where(byT == mby, iota, C), axis=0, keepdims=True)
        hit = (pred == lab).astype(jnp.float32)
        acc_ref[...] = jnp.mean(hit, axis=1, keepdims=True)


def kernel(bx, by, w1, b1, w2, b2, w, c, g, b, w3, b3, w4, b4):
    bxT = jnp.transpose(bx, (1, 0, 2))

    z_all = pl.pallas_call(
        _z_kernel,
        grid=(T, B // BC),
        in_specs=[
            pl.BlockSpec((1, BC, DIN), lambda t, i: (t, i, 0)),
            pl.BlockSpec((DIN, D1), lambda t, i: (0, 0)),
            pl.BlockSpec((1, D1), lambda t, i: (0, 0)),
            pl.BlockSpec((D1, H), lambda t, i: (0, 0)),
            pl.BlockSpec((1, H), lambda t, i: (0, 0)),
        ],
        out_specs=pl.BlockSpec((1, BC, H), lambda t, i: (t, i, 0)),
        out_shape=jax.ShapeDtypeStruct((T, B, H), jnp.float32),
        compiler_params=pltpu.CompilerParams(
            dimension_semantics=("parallel", "parallel"),
        ),
        name="z_precompute",
    )(bxT, w1, b1.reshape(1, D1), w2, b2.reshape(1, H))

    zT_all = jnp.transpose(z_all, (0, 2, 1))
    g2 = jnp.broadcast_to(g[:, None], (H, B))
    bb2 = jnp.broadcast_to(b[:, None], (H, B))
    b3g = jnp.broadcast_to(b3[:, None], (D1, B))
    b4g = jnp.broadcast_to(b4[:, None], (C, B))
    tt = np.arange(T)
    invh = jnp.asarray(0.5 * LAM ** (-tt.astype(np.float64)), jnp.float32)
    pow2 = jnp.asarray(2.0 * LAM ** (2.0 * tt), jnp.float32)

    loss, acc = pl.pallas_call(
        _scan_kernel,
        grid=(T,),
        in_specs=[
            pl.BlockSpec(memory_space=pltpu.SMEM),
            pl.BlockSpec(memory_space=pltpu.SMEM),
            pl.BlockSpec((1, H, B), lambda t: (t, 0, 0)),
            pl.BlockSpec((H, H), lambda t: (0, 0)),
            pl.BlockSpec((H, H), lambda t: (0, 0)),
            pl.BlockSpec((H, B), lambda t: (0, 0)),
            pl.BlockSpec((H, B), lambda t: (0, 0)),
            pl.BlockSpec((C, B), lambda t: (0, 0)),
            pl.BlockSpec((D1, H), lambda t: (0, 0)),
            pl.BlockSpec((D1, B), lambda t: (0, 0)),
            pl.BlockSpec((C, D1), lambda t: (0, 0)),
            pl.BlockSpec((C, B), lambda t: (0, 0)),
        ],
        out_specs=[
            pl.BlockSpec((1, 1), lambda t: (0, 0)),
            pl.BlockSpec((1, 1), lambda t: (0, 0)),
        ],
        out_shape=[
            jax.ShapeDtypeStruct((1, 1), jnp.float32),
            jax.ShapeDtypeStruct((1, 1), jnp.float32),
        ],
        scratch_shapes=[
            pltpu.VMEM((H, B), jnp.float32),
            pltpu.VMEM((H, H, B), jnp.float32),
            pltpu.VMEM((H, B), jnp.float32),
            pltpu.VMEM((H, B), jnp.float32),
            pltpu.VMEM((H, B), jnp.float32),
        ],
        compiler_params=pltpu.CompilerParams(
            dimension_semantics=("arbitrary",),
            vmem_limit_bytes=56 * 1024 * 1024,
        ),
        name="fast_weights_scan",
    )(invh, pow2, zT_all, w.T, c.T, g2, bb2, by.T,
      w3.T, b3g, w4.T, b4g)

    return loss[0, 0], acc[0, 0]


# speedup vs baseline: 1.2435x; 1.2435x over previous
import jax
import jax.numpy as jnp
import numpy as np
from jax.experimental import pallas as pl
from jax.experimental.pallas import tpu as pltpu

B, T, DIN, D1, H, C = 2048, 50, 128, 256, 64, 64
LAM, ETA = 0.95, 0.5
BC = 512


def _z_kernel(bx_ref, w1_ref, b1_ref, w2_ref, b2_ref, z_ref):
    x = bx_ref[0]
    s1 = jnp.maximum(
        jnp.dot(x, w1_ref[...], preferred_element_type=jnp.float32)
        + b1_ref[...], 0.0)
    z = jnp.maximum(
        jnp.dot(s1, w2_ref[...], preferred_element_type=jnp.float32)
        + b2_ref[...], 0.0)
    z_ref[0] = z


def _scan_kernel(invh_ref, pow2_ref, zT_ref, wT_ref, cT_ref, g2_ref,
                 bb2_ref, byT_ref, w3T_ref, b3g_ref, w4T_ref, b4g_ref,
                 loss_ref, acc_ref, hT_s, a_s, hnT_s, ehn_s, r_s):
    t = pl.program_id(0)

    @pl.when(t == 0)
    def _():
        hT_s[...] = jnp.zeros_like(hT_s)

        def zero_body(i, _):
            a_s[i] = jnp.zeros((H, B), jnp.float32)
            return 0
        jax.lax.fori_loop(0, H, zero_body, 0)

    zT = zT_ref[0]
    hT_old = hT_s[...]
    zc = jnp.dot(cT_ref[...], zT, preferred_element_type=jnp.float32)
    hn = jnp.maximum(
        jnp.dot(wT_ref[...], hT_old, preferred_element_type=jnp.float32)
        + zc, 0.0)
    hnT_s[...] = hn
    ehn_s[...] = invh_ref[t] * hn
    r_s[...] = jnp.zeros_like(r_s)

    def tile_body(kt, _):
        o = pl.multiple_of(kt * 32, 32)
        et32 = ehn_s[pl.ds(o, 32), :]
        hv = hnT_s[...]
        acc = None
        for u in range(32):
            i1 = o + u
            ehr = jnp.broadcast_to(et32[u:u + 1, :], (H, B))
            ao = a_s[i1]
            av = ao + ehr * hv
            a_s[i1] = av
            contrib = ehr * av
            acc = contrib if acc is None else acc + contrib
        r_s[...] += acc
        return 0

    jax.lax.fori_loop(0, H // 32, tile_body, 0)

    hn2 = hnT_s[...]
    rT = pow2_ref[t] * r_s[...]
    hs = jnp.dot(wT_ref[...], hn2, preferred_element_type=jnp.float32) \
        + zc + rT
    mu = jnp.mean(hs, axis=1, keepdims=True)
    d = hs - mu
    sig = jnp.sqrt(jnp.mean(d * d, axis=1, keepdims=True))
    hs = jnp.maximum(g2_ref[...] * d / sig + bb2_ref[...], 0.0)
    hT_s[...] = hs

    @pl.when(t == T - 1)
    def _():
        hh = jnp.maximum(
            jnp.dot(w3T_ref[...], hs, preferred_element_type=jnp.float32)
            + b3g_ref[...], 0.0)
        logits = jnp.dot(w4T_ref[...], hh, preferred_element_type=jnp.float32) \
            + b4g_ref[...]
        m = jnp.max(logits, axis=0, keepdims=True)
        sh = logits - m
        lse = jnp.log(jnp.sum(jnp.exp(sh), axis=0, keepdims=True))
        lsm = sh - lse
        byT = byT_ref[...]
        per_sample = jnp.sum(-byT * lsm, axis=0, keepdims=True)
        loss_ref[...] = jnp.mean(per_sample, axis=1, keepdims=True)
        iota = jax.lax.broadcasted_iota(jnp.int32, (C, B), 0)
        pred = jnp.min(jnp.where(logits == m, iota, C), axis=0, keepdims=True)
        mby = jnp.max(byT, axis=0, keepdims=True)
        lab = jnp.min(jnp.where(byT == mby, iota, C), axis=0, keepdims=True)
        hit = (pred == lab).astype(jnp.float32)
        acc_ref[...] = jnp.mean(hit, axis=1, keepdims=True)


def kernel(bx, by, w1, b1, w2, b2, w, c, g, b, w3, b3, w4, b4):
    bxT = jnp.transpose(bx, (1, 0, 2))

    z_all = pl.pallas_call(
        _z_kernel,
        grid=(T, B // BC),
        in_specs=[
            pl.BlockSpec((1, BC, DIN), lambda t, i: (t, i, 0)),
            pl.BlockSpec((DIN, D1), lambda t, i: (0, 0)),
            pl.BlockSpec((1, D1), lambda t, i: (0, 0)),
            pl.BlockSpec((D1, H), lambda t, i: (0, 0)),
            pl.BlockSpec((1, H), lambda t, i: (0, 0)),
        ],
        out_specs=pl.BlockSpec((1, BC, H), lambda t, i: (t, i, 0)),
        out_shape=jax.ShapeDtypeStruct((T, B, H), jnp.float32),
        compiler_params=pltpu.CompilerParams(
            dimension_semantics=("parallel", "parallel"),
        ),
        name="z_precompute",
    )(bxT, w1, b1.reshape(1, D1), w2, b2.reshape(1, H))

    zT_all = jnp.transpose(z_all, (0, 2, 1))
    g2 = jnp.broadcast_to(g[:, None], (H, B))
    bb2 = jnp.broadcast_to(b[:, None], (H, B))
    b3g = jnp.broadcast_to(b3[:, None], (D1, B))
    b4g = jnp.broadcast_to(b4[:, None], (C, B))
    tt = np.arange(T)
    invh = jnp.asarray(0.5 * LAM ** (-tt.astype(np.float64)), jnp.float32)
    pow2 = jnp.asarray(2.0 * LAM ** (2.0 * tt), jnp.float32)

    loss, acc = pl.pallas_call(
        _scan_kernel,
        grid=(T,),
        in_specs=[
            pl.BlockSpec(memory_space=pltpu.SMEM),
            pl.BlockSpec(memory_space=pltpu.SMEM),
            pl.BlockSpec((1, H, B), lambda t: (t, 0, 0)),
            pl.BlockSpec((H, H), lambda t: (0, 0)),
            pl.BlockSpec((H, H), lambda t: (0, 0)),
            pl.BlockSpec((H, B), lambda t: (0, 0)),
            pl.BlockSpec((H, B), lambda t: (0, 0)),
            pl.BlockSpec((C, B), lambda t: (0, 0)),
            pl.BlockSpec((D1, H), lambda t: (0, 0)),
            pl.BlockSpec((D1, B), lambda t: (0, 0)),
            pl.BlockSpec((C, D1), lambda t: (0, 0)),
            pl.BlockSpec((C, B), lambda t: (0, 0)),
        ],
        out_specs=[
            pl.BlockSpec((1, 1), lambda t: (0, 0)),
            pl.BlockSpec((1, 1), lambda t: (0, 0)),
        ],
        out_shape=[
            jax.ShapeDtypeStruct((1, 1), jnp.float32),
            jax.ShapeDtypeStruct((1, 1), jnp.float32),
        ],
        scratch_shapes=[
            pltpu.VMEM((H, B), jnp.float32),
            pltpu.VMEM((H, H, B), jnp.float32),
            pltpu.VMEM((H, B), jnp.float32),
            pltpu.VMEM((H, B), jnp.float32),
            pltpu.VMEM((H, B), jnp.float32),
        ],
        compiler_params=pltpu.CompilerParams(
            dimension_semantics=("arbitrary",),
            vmem_limit_bytes=56 * 1024 * 1024,
        ),
        name="fast_weights_scan",
    )(invh, pow2, zT_all, w.T, c.T, g2, bb2, by.T,
      w3.T, b3g, w4.T, b4g)

    return loss[0, 0], acc[0, 0]


# speedup vs baseline: 1.3946x; 1.1215x over previous
import jax
import jax.numpy as jnp
import numpy as np
from jax.experimental import pallas as pl
from jax.experimental.pallas import tpu as pltpu

B, T, DIN, D1, H, C = 2048, 50, 128, 256, 64, 64
LAM, ETA = 0.95, 0.5
BC = 1024


def _z_kernel(bx_ref, w1_ref, b1_ref, w2_ref, b2_ref, z_ref):
    x = bx_ref[0]
    s1 = jnp.maximum(
        jnp.dot(x, w1_ref[...], preferred_element_type=jnp.float32)
        + b1_ref[...], 0.0)
    z = jnp.maximum(
        jnp.dot(s1, w2_ref[...], preferred_element_type=jnp.float32)
        + b2_ref[...], 0.0)
    z_ref[0] = z


def _scan_kernel(invh_ref, pow2_ref, zT_ref, wT_ref, cT_ref, g2_ref,
                 bb2_ref, byT_ref, w3T_ref, b3g_ref, w4T_ref, b4g_ref,
                 loss_ref, acc_ref, hT_s, a_s, hnT_s, ehn_s, r_s):
    t = pl.program_id(0)

    @pl.when(t == 0)
    def _():
        hT_s[...] = jnp.zeros_like(hT_s)

        def zero_body(i, _):
            a_s[i] = jnp.zeros((H, B), jnp.float32)
            return 0
        jax.lax.fori_loop(0, H, zero_body, 0)

    zT = zT_ref[0]
    hT_old = hT_s[...]
    zc = jnp.dot(cT_ref[...], zT, preferred_element_type=jnp.float32)
    hn = jnp.maximum(
        jnp.dot(wT_ref[...], hT_old, preferred_element_type=jnp.float32)
        + zc, 0.0)
    hnT_s[...] = hn
    ehn_s[...] = invh_ref[t] * hn
    r_s[...] = jnp.zeros_like(r_s)

    def tile_body(kt, _):
        o = pl.multiple_of(kt * 32, 32)
        et32 = ehn_s[pl.ds(o, 32), :]
        hv = hnT_s[...]
        acc = None
        for u in range(32):
            i1 = o + u
            ehr = jnp.broadcast_to(et32[u:u + 1, :], (H, B))
            ao = a_s[i1]
            av = ao + ehr * hv
            a_s[i1] = av
            contrib = ehr * av
            acc = contrib if acc is None else acc + contrib
        r_s[...] += acc
        return 0

    jax.lax.fori_loop(0, H // 32, tile_body, 0)

    hn2 = hnT_s[...]
    rT = pow2_ref[t] * r_s[...]
    hs = jnp.dot(wT_ref[...], hn2, preferred_element_type=jnp.float32) \
        + zc + rT
    mu = jnp.mean(hs, axis=1, keepdims=True)
    d = hs - mu
    sig = jnp.sqrt(jnp.mean(d * d, axis=1, keepdims=True))
    hs = jnp.maximum(g2_ref[...] * d / sig + bb2_ref[...], 0.0)
    hT_s[...] = hs

    @pl.when(t == T - 1)
    def _():
        hh = jnp.maximum(
            jnp.dot(w3T_ref[...], hs, preferred_element_type=jnp.float32)
            + b3g_ref[...], 0.0)
        logits = jnp.dot(w4T_ref[...], hh, preferred_element_type=jnp.float32) \
            + b4g_ref[...]
        m = jnp.max(logits, axis=0, keepdims=True)
        sh = logits - m
        lse = jnp.log(jnp.sum(jnp.exp(sh), axis=0, keepdims=True))
        lsm = sh - lse
        byT = byT_ref[...]
        per_sample = jnp.sum(-byT * lsm, axis=0, keepdims=True)
        loss_ref[...] = jnp.mean(per_sample, axis=1, keepdims=True)
        iota = jax.lax.broadcasted_iota(jnp.int32, (C, B), 0)
        pred = jnp.min(jnp.where(logits == m, iota, C), axis=0, keepdims=True)
        mby = jnp.max(byT, axis=0, keepdims=True)
        lab = jnp.min(jnp.where(byT == mby, iota, C), axis=0, keepdims=True)
        hit = (pred == lab).astype(jnp.float32)
        acc_ref[...] = jnp.mean(hit, axis=1, keepdims=True)


def kernel(bx, by, w1, b1, w2, b2, w, c, g, b, w3, b3, w4, b4):
    bxT = jnp.transpose(bx, (1, 0, 2))

    z_all = pl.pallas_call(
        _z_kernel,
        grid=(T, B // BC),
        in_specs=[
            pl.BlockSpec((1, BC, DIN), lambda t, i: (t, i, 0)),
            pl.BlockSpec((DIN, D1), lambda t, i: (0, 0)),
            pl.BlockSpec((1, D1), lambda t, i: (0, 0)),
            pl.BlockSpec((D1, H), lambda t, i: (0, 0)),
            pl.BlockSpec((1, H), lambda t, i: (0, 0)),
        ],
        out_specs=pl.BlockSpec((1, BC, H), lambda t, i: (t, i, 0)),
        out_shape=jax.ShapeDtypeStruct((T, B, H), jnp.float32),
        compiler_params=pltpu.CompilerParams(
            dimension_semantics=("parallel", "parallel"),
        ),
        name="z_precompute",
    )(bxT, w1, b1.reshape(1, D1), w2, b2.reshape(1, H))

    zT_all = jnp.transpose(z_all, (0, 2, 1))
    g2 = jnp.broadcast_to(g[:, None], (H, B))
    bb2 = jnp.broadcast_to(b[:, None], (H, B))
    b3g = jnp.broadcast_to(b3[:, None], (D1, B))
    b4g = jnp.broadcast_to(b4[:, None], (C, B))
    tt = np.arange(T)
    invh = jnp.asarray(0.5 * LAM ** (-tt.astype(np.float64)), jnp.float32)
    pow2 = jnp.asarray(2.0 * LAM ** (2.0 * tt), jnp.float32)

    loss, acc = pl.pallas_call(
        _scan_kernel,
        grid=(T,),
        in_specs=[
            pl.BlockSpec(memory_space=pltpu.SMEM),
            pl.BlockSpec(memory_space=pltpu.SMEM),
            pl.BlockSpec((1, H, B), lambda t: (t, 0, 0)),
            pl.BlockSpec((H, H), lambda t: (0, 0)),
            pl.BlockSpec((H, H), lambda t: (0, 0)),
            pl.BlockSpec((H, B), lambda t: (0, 0)),
            pl.BlockSpec((H, B), lambda t: (0, 0)),
            pl.BlockSpec((C, B), lambda t: (0, 0)),
            pl.BlockSpec((D1, H), lambda t: (0, 0)),
            pl.BlockSpec((D1, B), lambda t: (0, 0)),
            pl.BlockSpec((C, D1), lambda t: (0, 0)),
            pl.BlockSpec((C, B), lambda t: (0, 0)),
        ],
        out_specs=[
            pl.BlockSpec((1, 1), lambda t: (0, 0)),
            pl.BlockSpec((1, 1), lambda t: (0, 0)),
        ],
        out_shape=[
            jax.ShapeDtypeStruct((1, 1), jnp.float32),
            jax.ShapeDtypeStruct((1, 1), jnp.float32),
        ],
        scratch_shapes=[
            pltpu.VMEM((H, B), jnp.float32),
            pltpu.VMEM((H, H, B), jnp.float32),
            pltpu.VMEM((H, B), jnp.float32),
            pltpu.VMEM((H, B), jnp.float32),
            pltpu.VMEM((H, B), jnp.float32),
        ],
        compiler_params=pltpu.CompilerParams(
            dimension_semantics=("arbitrary",),
            vmem_limit_bytes=56 * 1024 * 1024,
            flags={"XLA_TPU_STORE_TO_LOAD_FORWARDING_WINDOW": 16384},
        ),
        name="fast_weights_scan",
    )(invh, pow2, zT_all, w.T, c.T, g2, bb2, by.T,
      w3.T, b3g, w4.T, b4g)

    return loss[0, 0], acc[0, 0]


# speedup vs baseline: 1.4128x; 1.0131x over previous
import jax
import jax.numpy as jnp
import numpy as np
from jax.experimental import pallas as pl
from jax.experimental.pallas import tpu as pltpu

B, T, DIN, D1, H, C = 2048, 50, 128, 256, 64, 64
LAM, ETA = 0.95, 0.5
BC = 1024


def _z_kernel(bx_ref, w1_ref, b1_ref, w2_ref, b2_ref, z_ref):
    x = bx_ref[0]
    s1 = jnp.maximum(
        jnp.dot(x, w1_ref[...], preferred_element_type=jnp.float32)
        + b1_ref[...], 0.0)
    z = jnp.maximum(
        jnp.dot(s1, w2_ref[...], preferred_element_type=jnp.float32)
        + b2_ref[...], 0.0)
    z_ref[0] = z


def _scan_kernel(invh_ref, pow2_ref, zT_ref, wT_ref, cT_ref, g2_ref,
                 bb2_ref, byT_ref, w3T_ref, b3g_ref, w4T_ref, b4g_ref,
                 loss_ref, acc_ref, hT_s, a_s, hnT_s, ehn_s, r_s):
    t = pl.program_id(0)

    @pl.when(t == 0)
    def _():
        hT_s[...] = jnp.zeros_like(hT_s)

        def zero_body(i, _):
            a_s[i] = jnp.zeros((H, B), jnp.float32)
            return 0
        jax.lax.fori_loop(0, H, zero_body, 0)

    zT = zT_ref[0]
    hT_old = hT_s[...]
    zc = jnp.dot(cT_ref[...], zT, preferred_element_type=jnp.float32)
    hn = jnp.maximum(
        jnp.dot(wT_ref[...], hT_old, preferred_element_type=jnp.float32)
        + zc, 0.0)
    hnT_s[...] = hn
    ehn_s[...] = invh_ref[t] * hn
    r_s[...] = jnp.zeros_like(r_s)

    def tile_body(kt, _):
        o = pl.multiple_of(kt * 32, 32)
        et32 = ehn_s[pl.ds(o, 32), :]
        hv = hnT_s[...]
        acc = None
        for u in range(32):
            i1 = o + u
            ehr = jnp.broadcast_to(et32[u:u + 1, :], (H, B))
            ao = a_s[i1]
            av = ao + ehr * hv
            a_s[i1] = av
            contrib = ehr * av
            acc = contrib if acc is None else acc + contrib
        r_s[...] += acc
        return 0

    jax.lax.fori_loop(0, H // 32, tile_body, 0)

    hn2 = hnT_s[...]
    rT = pow2_ref[t] * r_s[...]
    hs = jnp.dot(wT_ref[...], hn2, preferred_element_type=jnp.float32) \
        + zc + rT
    mu = jnp.mean(hs, axis=1, keepdims=True)
    d = hs - mu
    sig = jnp.sqrt(jnp.mean(d * d, axis=1, keepdims=True))
    hs = jnp.maximum(g2_ref[...] * d / sig + bb2_ref[...], 0.0)
    hT_s[...] = hs

    @pl.when(t == T - 1)
    def _():
        hh = jnp.maximum(
            jnp.dot(w3T_ref[...], hs, preferred_element_type=jnp.float32)
            + b3g_ref[...], 0.0)
        logits = jnp.dot(w4T_ref[...], hh, preferred_element_type=jnp.float32) \
            + b4g_ref[...]
        m = jnp.max(logits, axis=0, keepdims=True)
        sh = logits - m
        lse = jnp.log(jnp.sum(jnp.exp(sh), axis=0, keepdims=True))
        lsm = sh - lse
        byT = byT_ref[...]
        per_sample = jnp.sum(-byT * lsm, axis=0, keepdims=True)
        loss_ref[...] = jnp.mean(per_sample, axis=1, keepdims=True)
        iota = jax.lax.broadcasted_iota(jnp.int32, (C, B), 0)
        pred = jnp.min(jnp.where(logits == m, iota, C), axis=0, keepdims=True)
        mby = jnp.max(byT, axis=0, keepdims=True)
        lab = jnp.min(jnp.where(byT == mby, iota, C), axis=0, keepdims=True)
        hit = (pred == lab).astype(jnp.float32)
        acc_ref[...] = jnp.mean(hit, axis=1, keepdims=True)


def kernel(bx, by, w1, b1, w2, b2, w, c, g, b, w3, b3, w4, b4):
    bxT = jnp.transpose(bx, (1, 0, 2))

    z_all = pl.pallas_call(
        _z_kernel,
        grid=(T, B // BC),
        in_specs=[
            pl.BlockSpec((1, BC, DIN), lambda t, i: (t, i, 0)),
            pl.BlockSpec((DIN, D1), lambda t, i: (0, 0)),
            pl.BlockSpec((1, D1), lambda t, i: (0, 0)),
            pl.BlockSpec((D1, H), lambda t, i: (0, 0)),
            pl.BlockSpec((1, H), lambda t, i: (0, 0)),
        ],
        out_specs=pl.BlockSpec((1, BC, H), lambda t, i: (t, i, 0)),
        out_shape=jax.ShapeDtypeStruct((T, B, H), jnp.float32),
        compiler_params=pltpu.CompilerParams(
            dimension_semantics=("parallel", "parallel"),
        ),
        name="z_precompute",
    )(bxT, w1, b1.reshape(1, D1), w2, b2.reshape(1, H))

    zT_all = jnp.transpose(z_all, (0, 2, 1))
    g2 = jnp.broadcast_to(g[:, None], (H, B))
    bb2 = jnp.broadcast_to(b[:, None], (H, B))
    b3g = jnp.broadcast_to(b3[:, None], (D1, B))
    b4g = jnp.broadcast_to(b4[:, None], (C, B))
    tt = np.arange(T)
    invh = jnp.asarray(0.5 * LAM ** (-tt.astype(np.float64)), jnp.float32)
    pow2 = jnp.asarray(2.0 * LAM ** (2.0 * tt), jnp.float32)

    loss, acc = pl.pallas_call(
        _scan_kernel,
        grid=(T,),
        in_specs=[
            pl.BlockSpec(memory_space=pltpu.SMEM),
            pl.BlockSpec(memory_space=pltpu.SMEM),
            pl.BlockSpec((1, H, B), lambda t: (t, 0, 0)),
            pl.BlockSpec((H, H), lambda t: (0, 0)),
            pl.BlockSpec((H, H), lambda t: (0, 0)),
            pl.BlockSpec((H, B), lambda t: (0, 0)),
            pl.BlockSpec((H, B), lambda t: (0, 0)),
            pl.BlockSpec((C, B), lambda t: (0, 0)),
            pl.BlockSpec((D1, H), lambda t: (0, 0)),
            pl.BlockSpec((D1, B), lambda t: (0, 0)),
            pl.BlockSpec((C, D1), lambda t: (0, 0)),
            pl.BlockSpec((C, B), lambda t: (0, 0)),
        ],
        out_specs=[
            pl.BlockSpec((1, 1), lambda t: (0, 0)),
            pl.BlockSpec((1, 1), lambda t: (0, 0)),
        ],
        out_shape=[
            jax.ShapeDtypeStruct((1, 1), jnp.float32),
            jax.ShapeDtypeStruct((1, 1), jnp.float32),
        ],
        scratch_shapes=[
            pltpu.VMEM((H, B), jnp.float32),
            pltpu.VMEM((H, H, B), jnp.float32),
            pltpu.VMEM((H, B), jnp.float32),
            pltpu.VMEM((H, B), jnp.float32),
            pltpu.VMEM((H, B), jnp.float32),
        ],
        compiler_params=pltpu.CompilerParams(
            dimension_semantics=("arbitrary",),
            vmem_limit_bytes=56 * 1024 * 1024,
            flags={"XLA_TPU_STORE_TO_LOAD_FORWARDING_WINDOW": 32768},
        ),
        name="fast_weights_scan",
    )(invh, pow2, zT_all, w.T, c.T, g2, bb2, by.T,
      w3.T, b3g, w4.T, b4g)

    return loss[0, 0], acc[0, 0]
